# baseline (speedup 1.0000x reference)
import functools

import jax
import jax.numpy as jnp
from jax import lax
from jax.experimental import pallas as pl
from jax.experimental.pallas import tpu as pltpu
from jax.experimental.pallas import tpu_sc as plsc

N = 10000
E = 320000
D = 128
NUM_GRAPHS = 64

NC = 2
NS = 16
NW = NC * NS
G = 128

NG0 = 40
NG1 = 120
EPT0 = NG0 * G
EPT1 = NG1 * G
C1_OFF = NS * EPT0
E_PAD = NS * (EPT0 + EPT1)
ROWS_PT = 632
N_ACC = NS * ROWS_PT
TAIL = N - 15 * ROWS_PT
TAIL_Z = N_ACC - N

_mesh = plsc.VectorSubcoreMesh(core_axis_name="c", subcore_axis_name="s")


@functools.partial(
    pl.kernel,
    mesh=_mesh,
    out_type=jax.ShapeDtypeStruct((NC, N, D), jnp.float32),
    scratch_types=[
        pltpu.VMEM((G,), jnp.int32),
        pltpu.VMEM((G,), jnp.int32),
        pltpu.VMEM((G, D), jnp.float32),
        pltpu.VMEM_SHARED((N_ACC, D), jnp.float32),
        pltpu.SemaphoreType.DMA,
    ],
)
def _sc_aggregate(x_hbm, src_hbm, dst_hbm, zrows_hbm, out_hbm,
                  idx_s, idx_d, rows, acc, sem):
    cid = lax.axis_index("c")
    sid = lax.axis_index("s")
    row0 = sid * ROWS_PT
    ebase = jnp.where(cid == 0, sid * EPT0, C1_OFF + sid * EPT1)
    n_groups = jnp.where(cid == 0, NG0, NG1)

    @pl.when(jnp.logical_and(cid == 0, sid < NS - 1))
    def _():
        pltpu.sync_copy(x_hbm.at[pl.ds(row0, ROWS_PT)],
                        acc.at[pl.ds(row0, ROWS_PT)])

    @pl.when(jnp.logical_and(cid == 0, sid == NS - 1))
    def _():
        pltpu.sync_copy(x_hbm.at[pl.ds(15 * ROWS_PT, TAIL)],
                        acc.at[pl.ds(15 * ROWS_PT, TAIL)])
        pltpu.sync_copy(zrows_hbm.at[pl.ds(0, TAIL_Z)],
                        acc.at[pl.ds(N, TAIL_Z)])

    @pl.when(cid != 0)
    def _():
        pltpu.sync_copy(zrows_hbm, acc.at[pl.ds(row0, ROWS_PT)])

    plsc.subcore_barrier()

    def body(g, carry):
        base = ebase + g * G
        pltpu.sync_copy(src_hbm.at[pl.ds(base, G)], idx_s)
        pltpu.sync_copy(dst_hbm.at[pl.ds(base, G)], idx_d)
        pltpu.async_copy(x_hbm.at[idx_s], rows, sem).wait()
        pltpu.sync_copy(rows, acc.at[idx_d], add=True)
        return carry

    lax.fori_loop(0, n_groups, body, 0)

    plsc.subcore_barrier()

    @pl.when(sid < NS - 1)
    def _():
        pltpu.sync_copy(acc.at[pl.ds(row0, ROWS_PT)],
                        out_hbm.at[cid, pl.ds(row0, ROWS_PT)])

    @pl.when(sid == NS - 1)
    def _():
        pltpu.sync_copy(acc.at[pl.ds(15 * ROWS_PT, TAIL)],
                        out_hbm.at[cid, pl.ds(15 * ROWS_PT, TAIL)])


ROWS_TC = 1000
N_BLOCKS = N // ROWS_TC


def _mlp_body(agg_ref, w1_ref, b1_ref, w2_ref, b2_ref):
    h = agg_ref[0] + agg_ref[1]
    h = jnp.maximum(
        jnp.dot(h, w1_ref[...], preferred_element_type=jnp.float32)
        + b1_ref[...], 0.0)
    h = jnp.maximum(
        jnp.dot(h, w2_ref[...], preferred_element_type=jnp.float32)
        + b2_ref[...], 0.0)
    return h


def _mlp_kernel(agg_ref, w1_ref, b1_ref, w2_ref, b2_ref, o_ref):
    o_ref[...] = _mlp_body(agg_ref, w1_ref, b1_ref, w2_ref, b2_ref)


def _mlp_pool_kernel(agg_ref, w1_ref, b1_ref, w2_ref, b2_ref, batch_ref,
                     o_ref, counts):
    i = pl.program_id(0)

    @pl.when(i == 0)
    def _():
        o_ref[...] = jnp.zeros_like(o_ref)
        counts[...] = jnp.zeros_like(counts)

    h = _mlp_body(agg_ref, w1_ref, b1_ref, w2_ref, b2_ref)
    b = batch_ref[0, 0, :]
    onehot = (b[:, None]
              == lax.broadcasted_iota(jnp.int32, (ROWS_TC, NUM_GRAPHS), 1)
              ).astype(jnp.float32)
    o_ref[...] += lax.dot_general(
        onehot, h, (((0,), (0,)), ((), ())),
        preferred_element_type=jnp.float32)
    counts[...] += jnp.sum(onehot, axis=0)[:, None]

    @pl.when(i == N_BLOCKS - 1)
    def _():
        o_ref[...] = o_ref[...] / jnp.maximum(counts[...], 1.0)


_w_spec = pl.BlockSpec((D, D), lambda i: (0, 0))
_b_spec = pl.BlockSpec((1, D), lambda i: (0, 0))
_agg_spec = pl.BlockSpec((NC, ROWS_TC, D), lambda i: (0, i, 0))


def _tc_mlp(agg, w1, b1, w2, b2):
    return pl.pallas_call(
        _mlp_kernel,
        grid=(N_BLOCKS,),
        in_specs=[_agg_spec, _w_spec, _b_spec, _w_spec, _b_spec],
        out_specs=pl.BlockSpec((ROWS_TC, D), lambda i: (i, 0)),
        out_shape=jax.ShapeDtypeStruct((N, D), jnp.float32),
    )(agg, w1, b1.reshape(1, D), w2, b2.reshape(1, D))


def _tc_mlp_pool(agg, w1, b1, w2, b2, batch_r):
    return pl.pallas_call(
        _mlp_pool_kernel,
        grid=(N_BLOCKS,),
        in_specs=[_agg_spec, _w_spec, _b_spec, _w_spec, _b_spec,
                  pl.BlockSpec((1, 1, ROWS_TC), lambda i: (i, 0, 0))],
        out_specs=pl.BlockSpec((NUM_GRAPHS, D), lambda i: (0, 0)),
        out_shape=jax.ShapeDtypeStruct((NUM_GRAPHS, D), jnp.float32),
        scratch_shapes=[pltpu.VMEM((NUM_GRAPHS, D), jnp.float32)],
    )(agg, w1, b1.reshape(1, D), w2, b2.reshape(1, D), batch_r)


@jax.jit
def kernel(x, edge_index, batch, W1a, b1a, W2a, b2a, W1b, b1b, W2b, b2b):
    pad = E_PAD - E
    src = jnp.concatenate([edge_index[0], jnp.zeros((pad,), jnp.int32)])
    dst = jnp.concatenate([edge_index[1], jnp.full((pad,), N, jnp.int32)])
    zrows = jnp.zeros((ROWS_PT, D), jnp.float32)
    batch_r = batch.reshape(N_BLOCKS, 1, ROWS_TC)

    agg1 = _sc_aggregate(x, src, dst, zrows)
    h1 = _tc_mlp(agg1, W1a, b1a, W2a, b2a)
    agg2 = _sc_aggregate(h1, src, dst, zrows)
    return _tc_mlp_pool(agg2, W1b, b1b, W2b, b2b, batch_r)


# speedup vs baseline: 1.2688x; 1.2688x over previous
import functools

import jax
import jax.numpy as jnp
from jax import lax
from jax.experimental import pallas as pl
from jax.experimental.pallas import tpu as pltpu
from jax.experimental.pallas import tpu_sc as plsc

N = 10000
E = 320000
D = 128
NUM_GRAPHS = 64

NC = 2
NS = 16
NW = NC * NS
G = 128

NG0 = 96
NG1 = 64
EPT0 = NG0 * G
EPT1 = NG1 * G
C1_OFF = NS * EPT0
E_PAD = NS * (EPT0 + EPT1)
ROWS_PT = 632
N_ACC = NS * ROWS_PT
TAIL = N - 15 * ROWS_PT
TAIL_Z = N_ACC - N

_mesh = plsc.VectorSubcoreMesh(core_axis_name="c", subcore_axis_name="s")


@functools.partial(
    pl.kernel,
    mesh=_mesh,
    out_type=jax.ShapeDtypeStruct((NC, N, D), jnp.float32),
    scratch_types=[
        pltpu.VMEM((G,), jnp.int32),
        pltpu.VMEM((G,), jnp.int32),
        pltpu.VMEM((G, D), jnp.float32),
        pltpu.VMEM_SHARED((N_ACC, D), jnp.float32),
        pltpu.SemaphoreType.DMA,
    ],
)
def _sc_aggregate(x_hbm, src_hbm, dst_hbm, zrows_hbm, out_hbm,
                  idx_s, idx_d, rows, acc, sem):
    cid = lax.axis_index("c")
    sid = lax.axis_index("s")
    row0 = sid * ROWS_PT
    ebase = jnp.where(cid == 0, sid * EPT0, C1_OFF + sid * EPT1)
    n_groups = jnp.where(cid == 0, NG0, NG1)

    @pl.when(jnp.logical_and(cid == 0, sid < NS - 1))
    def _():
        pltpu.sync_copy(x_hbm.at[pl.ds(row0, ROWS_PT)],
                        acc.at[pl.ds(row0, ROWS_PT)])

    @pl.when(jnp.logical_and(cid == 0, sid == NS - 1))
    def _():
        pltpu.sync_copy(x_hbm.at[pl.ds(15 * ROWS_PT, TAIL)],
                        acc.at[pl.ds(15 * ROWS_PT, TAIL)])
        pltpu.sync_copy(zrows_hbm.at[pl.ds(0, TAIL_Z)],
                        acc.at[pl.ds(N, TAIL_Z)])

    @pl.when(cid != 0)
    def _():
        pltpu.sync_copy(zrows_hbm, acc.at[pl.ds(row0, ROWS_PT)])

    plsc.subcore_barrier()

    def body(g, carry):
        base = ebase + g * G
        pltpu.sync_copy(src_hbm.at[pl.ds(base, G)], idx_s)
        pltpu.sync_copy(dst_hbm.at[pl.ds(base, G)], idx_d)
        pltpu.async_copy(x_hbm.at[idx_s], rows, sem).wait()
        pltpu.sync_copy(rows, acc.at[idx_d], add=True)
        return carry

    lax.fori_loop(0, n_groups, body, 0)

    plsc.subcore_barrier()

    @pl.when(sid < NS - 1)
    def _():
        pltpu.sync_copy(acc.at[pl.ds(row0, ROWS_PT)],
                        out_hbm.at[cid, pl.ds(row0, ROWS_PT)])

    @pl.when(sid == NS - 1)
    def _():
        pltpu.sync_copy(acc.at[pl.ds(15 * ROWS_PT, TAIL)],
                        out_hbm.at[cid, pl.ds(15 * ROWS_PT, TAIL)])


ROWS_TC = 1000
N_BLOCKS = N // ROWS_TC


def _mlp_body(agg_ref, w1_ref, b1_ref, w2_ref, b2_ref):
    h = agg_ref[0] + agg_ref[1]
    h = jnp.maximum(
        jnp.dot(h, w1_ref[...], preferred_element_type=jnp.float32)
        + b1_ref[...], 0.0)
    h = jnp.maximum(
        jnp.dot(h, w2_ref[...], preferred_element_type=jnp.float32)
        + b2_ref[...], 0.0)
    return h


def _mlp_kernel(agg_ref, w1_ref, b1_ref, w2_ref, b2_ref, o_ref):
    o_ref[...] = _mlp_body(agg_ref, w1_ref, b1_ref, w2_ref, b2_ref)


def _mlp_pool_kernel(agg_ref, w1_ref, b1_ref, w2_ref, b2_ref, batch_ref,
                     o_ref, counts):
    i = pl.program_id(0)

    @pl.when(i == 0)
    def _():
        o_ref[...] = jnp.zeros_like(o_ref)
        counts[...] = jnp.zeros_like(counts)

    h = _mlp_body(agg_ref, w1_ref, b1_ref, w2_ref, b2_ref)
    b = batch_ref[0, 0, :]
    onehot = (b[:, None]
              == lax.broadcasted_iota(jnp.int32, (ROWS_TC, NUM_GRAPHS), 1)
              ).astype(jnp.float32)
    o_ref[...] += lax.dot_general(
        onehot, h, (((0,), (0,)), ((), ())),
        preferred_element_type=jnp.float32)
    counts[...] += jnp.sum(onehot, axis=0)[:, None]

    @pl.when(i == N_BLOCKS - 1)
    def _():
        o_ref[...] = o_ref[...] / jnp.maximum(counts[...], 1.0)


_w_spec = pl.BlockSpec((D, D), lambda i: (0, 0))
_b_spec = pl.BlockSpec((1, D), lambda i: (0, 0))
_agg_spec = pl.BlockSpec((NC, ROWS_TC, D), lambda i: (0, i, 0))


def _tc_mlp(agg, w1, b1, w2, b2):
    return pl.pallas_call(
        _mlp_kernel,
        grid=(N_BLOCKS,),
        in_specs=[_agg_spec, _w_spec, _b_spec, _w_spec, _b_spec],
        out_specs=pl.BlockSpec((ROWS_TC, D), lambda i: (i, 0)),
        out_shape=jax.ShapeDtypeStruct((N, D), jnp.float32),
    )(agg, w1, b1.reshape(1, D), w2, b2.reshape(1, D))


def _tc_mlp_pool(agg, w1, b1, w2, b2, batch_r):
    return pl.pallas_call(
        _mlp_pool_kernel,
        grid=(N_BLOCKS,),
        in_specs=[_agg_spec, _w_spec, _b_spec, _w_spec, _b_spec,
                  pl.BlockSpec((1, 1, ROWS_TC), lambda i: (i, 0, 0))],
        out_specs=pl.BlockSpec((NUM_GRAPHS, D), lambda i: (0, 0)),
        out_shape=jax.ShapeDtypeStruct((NUM_GRAPHS, D), jnp.float32),
        scratch_shapes=[pltpu.VMEM((NUM_GRAPHS, D), jnp.float32)],
    )(agg, w1, b1.reshape(1, D), w2, b2.reshape(1, D), batch_r)


@jax.jit
def kernel(x, edge_index, batch, W1a, b1a, W2a, b2a, W1b, b1b, W2b, b2b):
    pad = E_PAD - E
    src = jnp.concatenate([edge_index[0], jnp.zeros((pad,), jnp.int32)])
    dst = jnp.concatenate([edge_index[1], jnp.full((pad,), N, jnp.int32)])
    zrows = jnp.zeros((ROWS_PT, D), jnp.float32)
    batch_r = batch.reshape(N_BLOCKS, 1, ROWS_TC)

    agg1 = _sc_aggregate(x, src, dst, zrows)
    h1 = _tc_mlp(agg1, W1a, b1a, W2a, b2a)
    agg2 = _sc_aggregate(h1, src, dst, zrows)
    return _tc_mlp_pool(agg2, W1b, b1b, W2b, b2b, batch_r)


# speedup vs baseline: 1.6241x; 1.2800x over previous
import functools

import jax
import jax.numpy as jnp
from jax import lax
from jax.experimental import pallas as pl
from jax.experimental.pallas import tpu as pltpu
from jax.experimental.pallas import tpu_sc as plsc

N = 10000
E = 320000
D = 128
NUM_GRAPHS = 64

NC = 2
NS = 16
NW = NC * NS
G = 128

EPG = NW * G
E_PAD = ((E + EPG - 1) // EPG) * EPG
EPT = E_PAD // NW
N_GROUPS = EPT // G
ROWS_PT = 632
N_ACC = NS * ROWS_PT
TAIL = N - 15 * ROWS_PT
TAIL_Z = N_ACC - N

_mesh = plsc.VectorSubcoreMesh(core_axis_name="c", subcore_axis_name="s")


@functools.partial(
    pl.kernel,
    mesh=_mesh,
    out_type=jax.ShapeDtypeStruct((NC, N, D), jnp.float32),
    scratch_types=[
        pltpu.VMEM((G,), jnp.int32),
        pltpu.VMEM((G,), jnp.int32),
        pltpu.VMEM((G, D), jnp.float32),
        pltpu.VMEM_SHARED((N_ACC, D), jnp.float32),
        pltpu.SemaphoreType.DMA,
    ],
)
def _sc_aggregate(x_hbm, src_hbm, dst_hbm, zrows_hbm, out_hbm,
                  idx_s, idx_d, rows, acc, sem):
    cid = lax.axis_index("c")
    sid = lax.axis_index("s")
    wid = sid * NC + cid
    row0 = sid * ROWS_PT

    @pl.when(jnp.logical_and(cid == 0, sid < NS - 1))
    def _():
        pltpu.sync_copy(x_hbm.at[pl.ds(row0, ROWS_PT)],
                        acc.at[pl.ds(row0, ROWS_PT)])

    @pl.when(jnp.logical_and(cid == 0, sid == NS - 1))
    def _():
        pltpu.sync_copy(x_hbm.at[pl.ds(15 * ROWS_PT, TAIL)],
                        acc.at[pl.ds(15 * ROWS_PT, TAIL)])
        pltpu.sync_copy(zrows_hbm.at[pl.ds(0, TAIL_Z)],
                        acc.at[pl.ds(N, TAIL_Z)])

    @pl.when(cid != 0)
    def _():
        pltpu.sync_copy(zrows_hbm, acc.at[pl.ds(row0, ROWS_PT)])

    plsc.subcore_barrier()

    def body(g, carry):
        base = wid * EPT + g * G
        pltpu.sync_copy(src_hbm.at[pl.ds(base, G)], idx_s)
        pltpu.sync_copy(dst_hbm.at[pl.ds(base, G)], idx_d)
        pltpu.async_copy(x_hbm.at[idx_s], rows, sem).wait()
        pltpu.sync_copy(rows, acc.at[idx_d], add=True)
        return carry

    lax.fori_loop(0, N_GROUPS, body, 0)

    plsc.subcore_barrier()

    @pl.when(sid < NS - 1)
    def _():
        pltpu.sync_copy(acc.at[pl.ds(row0, ROWS_PT)],
                        out_hbm.at[cid, pl.ds(row0, ROWS_PT)])

    @pl.when(sid == NS - 1)
    def _():
        pltpu.sync_copy(acc.at[pl.ds(15 * ROWS_PT, TAIL)],
                        out_hbm.at[cid, pl.ds(15 * ROWS_PT, TAIL)])


ROWS_TC = 1000
N_BLOCKS = N // ROWS_TC


def _mlp_body(agg_ref, w1_ref, b1_ref, w2_ref, b2_ref):
    h = agg_ref[0] + agg_ref[1]
    h = jnp.maximum(
        jnp.dot(h, w1_ref[...], preferred_element_type=jnp.float32)
        + b1_ref[...], 0.0)
    h = jnp.maximum(
        jnp.dot(h, w2_ref[...], preferred_element_type=jnp.float32)
        + b2_ref[...], 0.0)
    return h


def _mlp_kernel(agg_ref, w1_ref, b1_ref, w2_ref, b2_ref, o_ref):
    o_ref[...] = _mlp_body(agg_ref, w1_ref, b1_ref, w2_ref, b2_ref)


def _mlp_pool_kernel(agg_ref, w1_ref, b1_ref, w2_ref, b2_ref, batch_ref,
                     o_ref, counts):
    i = pl.program_id(0)

    @pl.when(i == 0)
    def _():
        o_ref[...] = jnp.zeros_like(o_ref)
        counts[...] = jnp.zeros_like(counts)

    h = _mlp_body(agg_ref, w1_ref, b1_ref, w2_ref, b2_ref)
    b = batch_ref[0, 0, :]
    onehot = (b[:, None]
              == lax.broadcasted_iota(jnp.int32, (ROWS_TC, NUM_GRAPHS), 1)
              ).astype(jnp.float32)
    o_ref[...] += lax.dot_general(
        onehot, h, (((0,), (0,)), ((), ())),
        preferred_element_type=jnp.float32)
    counts[...] += jnp.sum(onehot, axis=0)[:, None]

    @pl.when(i == N_BLOCKS - 1)
    def _():
        o_ref[...] = o_ref[...] / jnp.maximum(counts[...], 1.0)


_w_spec = pl.BlockSpec((D, D), lambda i: (0, 0))
_b_spec = pl.BlockSpec((1, D), lambda i: (0, 0))
_agg_spec = pl.BlockSpec((NC, ROWS_TC, D), lambda i: (0, i, 0))


def _tc_mlp(agg, w1, b1, w2, b2):
    return pl.pallas_call(
        _mlp_kernel,
        grid=(N_BLOCKS,),
        in_specs=[_agg_spec, _w_spec, _b_spec, _w_spec, _b_spec],
        out_specs=pl.BlockSpec((ROWS_TC, D), lambda i: (i, 0)),
        out_shape=jax.ShapeDtypeStruct((N, D), jnp.float32),
    )(agg, w1, b1.reshape(1, D), w2, b2.reshape(1, D))


def _tc_mlp_pool(agg, w1, b1, w2, b2, batch_r):
    return pl.pallas_call(
        _mlp_pool_kernel,
        grid=(N_BLOCKS,),
        in_specs=[_agg_spec, _w_spec, _b_spec, _w_spec, _b_spec,
                  pl.BlockSpec((1, 1, ROWS_TC), lambda i: (i, 0, 0))],
        out_specs=pl.BlockSpec((NUM_GRAPHS, D), lambda i: (0, 0)),
        out_shape=jax.ShapeDtypeStruct((NUM_GRAPHS, D), jnp.float32),
        scratch_shapes=[pltpu.VMEM((NUM_GRAPHS, D), jnp.float32)],
    )(agg, w1, b1.reshape(1, D), w2, b2.reshape(1, D), batch_r)


@jax.jit
def kernel(x, edge_index, batch, W1a, b1a, W2a, b2a, W1b, b1b, W2b, b2b):
    pad = E_PAD - E
    src = jnp.concatenate([edge_index[0], jnp.zeros((pad,), jnp.int32)])
    dst = jnp.concatenate([edge_index[1], jnp.full((pad,), N, jnp.int32)])
    zrows = jnp.zeros((ROWS_PT, D), jnp.float32)
    batch_r = batch.reshape(N_BLOCKS, 1, ROWS_TC)

    agg1 = _sc_aggregate(x, src, dst, zrows)
    h1 = _tc_mlp(agg1, W1a, b1a, W2a, b2a)
    agg2 = _sc_aggregate(h1, src, dst, zrows)
    return _tc_mlp_pool(agg2, W1b, b1b, W2b, b2b, batch_r)
